# baseline (speedup 1.0000x reference)
import functools

import jax
import jax.numpy as jnp
from jax import lax
from jax.experimental import pallas as pl
from jax.experimental.pallas import tpu as pltpu
from jax.experimental.pallas import tpu_sc as plsc

EB = 64
BATCH = 16384
NC = 2
NS = 16
NW = NC * NS
R = BATCH // NW
F = R * EB
NG = R // 16


def _sc_body(u_hbm, p_hbm, t_hbm, idx_hbm, b_hbm, table_hbm, out_hbm,
             u_v, p_v, t_v, idx_v, b_v, bias_v, out_v, sem):
    cid = lax.axis_index("c")
    sid = lax.axis_index("s")
    wid = sid * NC + cid
    base = wid * R

    pltpu.sync_copy(idx_hbm.at[pl.ds(base, R)], idx_v)
    gather = pltpu.async_copy(table_hbm.at[idx_v], bias_v, sem)

    pltpu.sync_copy(u_hbm.at[pl.ds(base * EB, F)], u_v)
    pltpu.sync_copy(p_hbm.at[pl.ds(base * EB, F)], p_v)
    pltpu.sync_copy(t_hbm.at[pl.ds(base * EB, F)], t_v)
    pltpu.sync_copy(b_hbm, b_v)
    gather.wait()

    b1 = [b_v[pl.ds(c * 16, 16)] for c in range(4)]
    b2 = [b_v[pl.ds(EB + c * 16, 16)] for c in range(4)]
    lane = lax.iota(jnp.int32, 16)

    def group(g, carry):
        goff = g * (16 * EB)
        tot = bias_v[pl.ds(g * 16, 16)]
        for r in range(16):
            off = goff + r * EB
            acc = None
            for c in range(4):
                uu = u_v[pl.ds(off + c * 16, 16)]
                pp = p_v[pl.ds(off + c * 16, 16)]
                tt = t_v[pl.ds(off + c * 16, 16)]
                d1 = (uu - tt) * b1[c]
                d2 = (pp - tt) * b2[c]
                term = d1 * d1 + d2 * d2
                acc = term if acc is None else acc + term
            tot = jnp.where(lane == r, tot + jnp.sum(acc), tot)
        out_v[pl.ds(g * 16, 16)] = tot
        return carry

    lax.fori_loop(0, NG, group, 0)
    pltpu.sync_copy(out_v, out_hbm.at[pl.ds(base, R)])


@jax.jit
def _mdr_sc(u, p, t, idx, b, table):
    mesh = plsc.VectorSubcoreMesh(core_axis_name="c", subcore_axis_name="s")
    call = functools.partial(
        pl.kernel,
        mesh=mesh,
        out_type=jax.ShapeDtypeStruct((BATCH,), jnp.float32),
        scratch_types=[
            pltpu.VMEM((F,), jnp.float32),
            pltpu.VMEM((F,), jnp.float32),
            pltpu.VMEM((F,), jnp.float32),
            pltpu.VMEM((R,), jnp.int32),
            pltpu.VMEM((2 * EB,), jnp.float32),
            pltpu.VMEM((R,), jnp.float32),
            pltpu.VMEM((R,), jnp.float32),
            pltpu.SemaphoreType.DMA,
        ],
        compiler_params=pltpu.CompilerParams(needs_layout_passes=False),
    )
    return call(_sc_body)(u, p, t, idx, b, table)


def kernel(user_ebs, playlist_ebs, track_ebs, track_entity_ids, B1, B2,
           track_biases):
    u = user_ebs.reshape(-1)
    p = playlist_ebs.reshape(-1)
    t = track_ebs.reshape(-1)
    idx = track_entity_ids.astype(jnp.int32)
    b = jnp.concatenate([B1, B2])
    return _mdr_sc(u, p, t, idx, b, track_biases)


# speedup vs baseline: 1.0545x; 1.0545x over previous
import functools

import jax
import jax.numpy as jnp
from jax import lax
from jax.experimental import pallas as pl
from jax.experimental.pallas import tpu as pltpu
from jax.experimental.pallas import tpu_sc as plsc

EB = 64
BATCH = 16384
NC = 2
NS = 16
NW = NC * NS
R = BATCH // NW
NG = R // 16


def _sc_body(u_hbm, p_hbm, t_hbm, idx_hbm, b1_hbm, b2_hbm, table_hbm,
             out_hbm, u_v, p_v, t_v, idx_v, b1_v, b2_v, bias_v, out_v,
             sem, semu, semp, semt):
    cid = lax.axis_index("c")
    sid = lax.axis_index("s")
    wid = sid * NC + cid
    base = wid * R

    pltpu.sync_copy(idx_hbm.at[pl.ds(base, R)], idx_v)
    gather = pltpu.async_copy(table_hbm.at[idx_v], bias_v, sem)

    cpu = pltpu.async_copy(u_hbm.at[pl.ds(base, R)], u_v, semu)
    cpp = pltpu.async_copy(p_hbm.at[pl.ds(base, R)], p_v, semp)
    cpt = pltpu.async_copy(t_hbm.at[pl.ds(base, R)], t_v, semt)
    pltpu.sync_copy(b1_hbm, b1_v)
    pltpu.sync_copy(b2_hbm, b2_v)
    gather.wait()
    cpu.wait()
    cpp.wait()
    cpt.wait()

    b1 = [b1_v[pl.ds(c * 16, 16)] for c in range(4)]
    b2 = [b2_v[pl.ds(c * 16, 16)] for c in range(4)]
    lane = lax.iota(jnp.int32, 16)

    def group(g, carry):
        row0 = g * 16
        tot = bias_v[pl.ds(row0, 16)]
        for r in range(16):
            acc = None
            for c in range(4):
                uu = u_v[row0 + r, pl.ds(c * 16, 16)]
                pp = p_v[row0 + r, pl.ds(c * 16, 16)]
                tt = t_v[row0 + r, pl.ds(c * 16, 16)]
                d1 = (uu - tt) * b1[c]
                d2 = (pp - tt) * b2[c]
                term = d1 * d1 + d2 * d2
                acc = term if acc is None else acc + term
            tot = jnp.where(lane == r, tot + jnp.sum(acc), tot)
        out_v[pl.ds(row0, 16)] = tot
        return carry

    lax.fori_loop(0, NG, group, 0)
    pltpu.sync_copy(out_v, out_hbm.at[pl.ds(base, R)])


@jax.jit
def _mdr_sc(u, p, t, idx, b1, b2, table):
    mesh = plsc.VectorSubcoreMesh(core_axis_name="c", subcore_axis_name="s")
    call = functools.partial(
        pl.kernel,
        mesh=mesh,
        out_type=jax.ShapeDtypeStruct((BATCH,), jnp.float32),
        scratch_types=[
            pltpu.VMEM((R, EB), jnp.float32),
            pltpu.VMEM((R, EB), jnp.float32),
            pltpu.VMEM((R, EB), jnp.float32),
            pltpu.VMEM((R,), jnp.int32),
            pltpu.VMEM((EB,), jnp.float32),
            pltpu.VMEM((EB,), jnp.float32),
            pltpu.VMEM((R,), jnp.float32),
            pltpu.VMEM((R,), jnp.float32),
            pltpu.SemaphoreType.DMA,
            pltpu.SemaphoreType.DMA,
            pltpu.SemaphoreType.DMA,
            pltpu.SemaphoreType.DMA,
        ],
        compiler_params=pltpu.CompilerParams(
            needs_layout_passes=False, use_tc_tiling_on_sc=False),
    )
    return call(_sc_body)(u, p, t, idx, b1, b2, table)


def kernel(user_ebs, playlist_ebs, track_ebs, track_entity_ids, B1, B2,
           track_biases):
    idx = track_entity_ids.astype(jnp.int32)
    return _mdr_sc(user_ebs, playlist_ebs, track_ebs, idx, B1, B2,
                   track_biases)


# speedup vs baseline: 1.3923x; 1.3204x over previous
import functools

import jax
import jax.numpy as jnp
from jax import lax
from jax.experimental import pallas as pl
from jax.experimental.pallas import tpu as pltpu
from jax.experimental.pallas import tpu_sc as plsc

EB = 64
BATCH = 16384
NC = 2
NS = 16
NW = NC * NS
R = BATCH // NW
NT = R // 128
LANE = 16


def _sc_body(u_hbm, p_hbm, t_hbm, idx_hbm, b1_hbm, b2_hbm, table_hbm,
             out_hbm, u_v, p_v, t_v, idx_v, b1_v, b2_v, bias_v, out_v,
             sem, semu, semp, semt):
    cid = lax.axis_index("c")
    sid = lax.axis_index("s")
    wid = sid * NC + cid
    base = wid * R

    pltpu.sync_copy(idx_hbm.at[pl.ds(base, R)], idx_v)
    gather = pltpu.async_copy(table_hbm.at[idx_v], bias_v, sem)

    cpu = pltpu.async_copy(u_hbm.at[:, pl.ds(base, R)], u_v, semu)
    cpp = pltpu.async_copy(p_hbm.at[:, pl.ds(base, R)], p_v, semp)
    cpt = pltpu.async_copy(t_hbm.at[:, pl.ds(base, R)], t_v, semt)
    pltpu.sync_copy(b1_hbm, b1_v)
    pltpu.sync_copy(b2_hbm, b2_v)
    gather.wait()
    cpu.wait()
    cpp.wait()
    cpt.wait()

    def tile(it, carry):
        for g8 in range(8):
            i0 = it * 128 + g8 * LANE
            tot = bias_v[pl.ds(it * 128 + g8 * LANE, LANE)]
            for j in range(EB):
                uu = u_v[j, pl.ds(i0, LANE)]
                pp = p_v[j, pl.ds(i0, LANE)]
                tt = t_v[j, pl.ds(i0, LANE)]
                d1 = (uu - tt) * b1_v[j, pl.ds(0, LANE)]
                d2 = (pp - tt) * b2_v[j, pl.ds(0, LANE)]
                tot = tot + d1 * d1 + d2 * d2
            out_v[pl.ds(i0, LANE)] = tot
        return carry

    lax.fori_loop(0, NT, tile, 0)
    pltpu.sync_copy(out_v, out_hbm.at[pl.ds(base, R)])


@jax.jit
def _mdr_sc(u, p, t, idx, b1, b2, table):
    mesh = plsc.VectorSubcoreMesh(core_axis_name="c", subcore_axis_name="s")
    call = functools.partial(
        pl.kernel,
        mesh=mesh,
        out_type=jax.ShapeDtypeStruct((BATCH,), jnp.float32),
        scratch_types=[
            pltpu.VMEM((EB, R), jnp.float32),
            pltpu.VMEM((EB, R), jnp.float32),
            pltpu.VMEM((EB, R), jnp.float32),
            pltpu.VMEM((R,), jnp.int32),
            pltpu.VMEM((EB, 128), jnp.float32),
            pltpu.VMEM((EB, 128), jnp.float32),
            pltpu.VMEM((R,), jnp.float32),
            pltpu.VMEM((R,), jnp.float32),
            pltpu.SemaphoreType.DMA,
            pltpu.SemaphoreType.DMA,
            pltpu.SemaphoreType.DMA,
            pltpu.SemaphoreType.DMA,
        ],
        compiler_params=pltpu.CompilerParams(
            needs_layout_passes=False, use_tc_tiling_on_sc=True),
    )
    return call(_sc_body)(u, p, t, idx, b1, b2, table)


def kernel(user_ebs, playlist_ebs, track_ebs, track_entity_ids, B1, B2,
           track_biases):
    u = user_ebs.T
    p = playlist_ebs.T
    t = track_ebs.T
    idx = track_entity_ids.astype(jnp.int32)
    b1 = jnp.broadcast_to(B1[:, None], (EB, 128))
    b2 = jnp.broadcast_to(B2[:, None], (EB, 128))
    return _mdr_sc(u, p, t, idx, b1, b2, track_biases)


# speedup vs baseline: 1.4662x; 1.0531x over previous
import functools

import jax
import jax.numpy as jnp
from jax import lax
from jax.experimental import pallas as pl
from jax.experimental.pallas import tpu as pltpu
from jax.experimental.pallas import tpu_sc as plsc

EB = 64
BATCH = 16384
NC = 2
NS = 16
NW = NC * NS
R = BATCH // NW
H = R // 2
LANE = 16
GPB = 8
NQ = H // (LANE * GPB)


def _sc_body(u_hbm, p_hbm, t_hbm, idx_hbm, b1_hbm, b2_hbm, table_hbm,
             out_hbm, u_v, p_v, t_v, idx_v, b1_v, b2_v, bias_v, out_v,
             sem, sems):
    cid = lax.axis_index("c")
    sid = lax.axis_index("s")
    wid = sid * NC + cid
    base = wid * R

    pltpu.sync_copy(idx_hbm.at[pl.ds(base, R)], idx_v)
    gather = pltpu.async_copy(table_hbm.at[idx_v], bias_v, sem)

    cps = []
    for h in range(2):
        off = base + h * H
        cps.append((
            pltpu.async_copy(u_hbm.at[:, pl.ds(off, H)],
                             u_v.at[:, pl.ds(h * H, H)], sems.at[h, 0]),
            pltpu.async_copy(p_hbm.at[:, pl.ds(off, H)],
                             p_v.at[:, pl.ds(h * H, H)], sems.at[h, 1]),
            pltpu.async_copy(t_hbm.at[:, pl.ds(off, H)],
                             t_v.at[:, pl.ds(h * H, H)], sems.at[h, 2]),
        ))
    pltpu.sync_copy(b1_hbm, b1_v)
    pltpu.sync_copy(b2_hbm, b2_v)
    gather.wait()

    for h in range(2):
        for cp in cps[h]:
            cp.wait()
        for j_hi in range(8):
            b1s = [b1_v[j_hi * 8 + j_lo, pl.ds(0, LANE)] for j_lo in range(8)]
            b2s = [b2_v[j_hi * 8 + j_lo, pl.ds(0, LANE)] for j_lo in range(8)]
            first = j_hi == 0

            def quad(q, carry):
                for gi in range(GPB):
                    i0 = h * H + (q * GPB + gi) * LANE
                    seed = (bias_v[pl.ds(i0, LANE)] if first
                            else out_v[pl.ds(i0, LANE)])
                    acc_u = seed
                    acc_p = None
                    for j_lo in range(8):
                        j = j_hi * 8 + j_lo
                        uu = u_v[j, pl.ds(i0, LANE)]
                        pp = p_v[j, pl.ds(i0, LANE)]
                        tt = t_v[j, pl.ds(i0, LANE)]
                        d1 = (uu - tt) * b1s[j_lo]
                        d2 = (pp - tt) * b2s[j_lo]
                        acc_u = acc_u + d1 * d1
                        acc_p = (d2 * d2 if acc_p is None
                                 else acc_p + d2 * d2)
                    out_v[pl.ds(i0, LANE)] = acc_u + acc_p
                return carry

            lax.fori_loop(0, NQ, quad, 0)

    pltpu.sync_copy(out_v, out_hbm.at[pl.ds(base, R)])


@jax.jit
def _mdr_sc(u, p, t, idx, b1, b2, table):
    mesh = plsc.VectorSubcoreMesh(core_axis_name="c", subcore_axis_name="s")
    call = functools.partial(
        pl.kernel,
        mesh=mesh,
        out_type=jax.ShapeDtypeStruct((BATCH,), jnp.float32),
        scratch_types=[
            pltpu.VMEM((EB, R), jnp.float32),
            pltpu.VMEM((EB, R), jnp.float32),
            pltpu.VMEM((EB, R), jnp.float32),
            pltpu.VMEM((R,), jnp.int32),
            pltpu.VMEM((EB, 128), jnp.float32),
            pltpu.VMEM((EB, 128), jnp.float32),
            pltpu.VMEM((R,), jnp.float32),
            pltpu.VMEM((R,), jnp.float32),
            pltpu.SemaphoreType.DMA,
            pltpu.SemaphoreType.DMA((2, 3)),
        ],
        compiler_params=pltpu.CompilerParams(
            needs_layout_passes=False, use_tc_tiling_on_sc=True),
    )
    return call(_sc_body)(u, p, t, idx, b1, b2, table)


def kernel(user_ebs, playlist_ebs, track_ebs, track_entity_ids, B1, B2,
           track_biases):
    u = user_ebs.T
    p = playlist_ebs.T
    t = track_ebs.T
    idx = track_entity_ids.astype(jnp.int32)
    b1 = jnp.broadcast_to(B1[:, None], (EB, 128))
    b2 = jnp.broadcast_to(B2[:, None], (EB, 128))
    return _mdr_sc(u, p, t, idx, b1, b2, track_biases)


# speedup vs baseline: 2.3612x; 1.6104x over previous
import functools

import jax
import jax.numpy as jnp
from jax import lax
from jax.experimental import pallas as pl
from jax.experimental.pallas import tpu as pltpu
from jax.experimental.pallas import tpu_sc as plsc

EB = 64
BATCH = 16384
NC = 2
NS = 16
NW = NC * NS
R = BATCH // NW
CB = 2048
JB = 4096


def _sc_gather_body(idx_hbm, table_hbm, out_hbm, idx_v, bias_v, sem):
    cid = lax.axis_index("c")
    sid = lax.axis_index("s")
    wid = sid * NC + cid
    base = wid * R
    pltpu.sync_copy(idx_hbm.at[pl.ds(base, R)], idx_v)
    pltpu.async_copy(table_hbm.at[idx_v], bias_v, sem).wait()
    pltpu.sync_copy(bias_v, out_hbm.at[pl.ds(base, R)])


@jax.jit
def _sc_gather(idx, table):
    mesh = plsc.VectorSubcoreMesh(core_axis_name="c", subcore_axis_name="s")
    call = functools.partial(
        pl.kernel,
        mesh=mesh,
        out_type=jax.ShapeDtypeStruct((BATCH,), jnp.float32),
        scratch_types=[
            pltpu.VMEM((R,), jnp.int32),
            pltpu.VMEM((R,), jnp.float32),
            pltpu.SemaphoreType.DMA,
        ],
        compiler_params=pltpu.CompilerParams(needs_layout_passes=False),
    )
    return call(_sc_gather_body)(idx, table)


def _tc_dense_body(u_ref, p_ref, t_ref, b1_ref, b2_ref, o_ref):
    u = u_ref[...]
    p = p_ref[...]
    t = t_ref[...]
    b1 = b1_ref[...]
    b2 = b2_ref[...]
    d1 = (u - t) * b1
    d2 = (p - t) * b2
    o_ref[...] = jnp.sum(d1 * d1 + d2 * d2, axis=0)


@jax.jit
def _tc_dense(u, p, t, b1, b2):
    grid = BATCH // CB
    return pl.pallas_call(
        _tc_dense_body,
        grid=(grid,),
        in_specs=[
            pl.BlockSpec((EB, CB), lambda i: (0, i)),
            pl.BlockSpec((EB, CB), lambda i: (0, i)),
            pl.BlockSpec((EB, CB), lambda i: (0, i)),
            pl.BlockSpec((EB, 1), lambda i: (0, 0)),
            pl.BlockSpec((EB, 1), lambda i: (0, 0)),
        ],
        out_specs=pl.BlockSpec((CB,), lambda i: (i,)),
        out_shape=jax.ShapeDtypeStruct((BATCH,), jnp.float32),
    )(u, p, t, b1, b2)


def _tc_join_body(a_ref, b_ref, o_ref):
    o_ref[...] = a_ref[...] + b_ref[...]


@jax.jit
def _tc_join(a, b):
    return pl.pallas_call(
        _tc_join_body,
        grid=(BATCH // JB,),
        in_specs=[
            pl.BlockSpec((JB,), lambda i: (i,)),
            pl.BlockSpec((JB,), lambda i: (i,)),
        ],
        out_specs=pl.BlockSpec((JB,), lambda i: (i,)),
        out_shape=jax.ShapeDtypeStruct((BATCH,), jnp.float32),
    )(a, b)


def kernel(user_ebs, playlist_ebs, track_ebs, track_entity_ids, B1, B2,
           track_biases):
    idx = track_entity_ids.astype(jnp.int32)
    bias = _sc_gather(idx, track_biases)
    o12 = _tc_dense(user_ebs.T, playlist_ebs.T, track_ebs.T,
                    B1[:, None], B2[:, None])
    return _tc_join(o12, bias)
